# Initial kernel scaffold; baseline (speedup 1.0000x reference)
import jax
import jax.numpy as jnp
from jax.experimental import pallas as pl


def kernel(x, emb):
    raise NotImplementedError("write your pallas kernel here")


# speedup vs baseline: 1.8302x; 1.8302x over previous
import jax
import jax.numpy as jnp
from jax.experimental import pallas as pl

_BS = 512


def _body(emb_ref, out_ref):
    out_ref[...] = emb_ref[...][None]


def kernel(x, emb):
    b, s, d = x.shape
    return pl.pallas_call(
        _body,
        grid=(s // _BS, b),
        in_specs=[pl.BlockSpec((_BS, d), lambda i, j: (i, 0))],
        out_specs=pl.BlockSpec((1, _BS, d), lambda i, j: (j, i, 0)),
        out_shape=jax.ShapeDtypeStruct((b, s, d), emb.dtype),
    )(emb)


# speedup vs baseline: 2.6783x; 1.4634x over previous
import jax
import jax.numpy as jnp
from jax.experimental import pallas as pl

_BS = 512


def _body(emb_ref, out_ref):
    out_ref[...] = jnp.broadcast_to(emb_ref[...][None], out_ref.shape)


def kernel(x, emb):
    b, s, d = x.shape
    return pl.pallas_call(
        _body,
        grid=(s // _BS,),
        in_specs=[pl.BlockSpec((_BS, d), lambda i: (i, 0))],
        out_specs=pl.BlockSpec((b, _BS, d), lambda i: (0, i, 0)),
        out_shape=jax.ShapeDtypeStruct((b, s, d), emb.dtype),
    )(emb)


# speedup vs baseline: 2.7602x; 1.0306x over previous
import jax
import jax.numpy as jnp
from jax.experimental import pallas as pl

_BS = 1024


def _body(emb_ref, out_ref):
    out_ref[...] = jnp.broadcast_to(emb_ref[...][None], out_ref.shape)


def kernel(x, emb):
    b, s, d = x.shape
    return pl.pallas_call(
        _body,
        grid=(s // _BS,),
        in_specs=[pl.BlockSpec((_BS, d), lambda i: (i, 0))],
        out_specs=pl.BlockSpec((b, _BS, d), lambda i: (0, i, 0)),
        out_shape=jax.ShapeDtypeStruct((b, s, d), emb.dtype),
    )(emb)
